# baseline (speedup 1.0000x reference)
import functools

import jax
import jax.numpy as jnp
from jax.experimental import pallas as pl

NUM_TOKENS = 16384
HIDDEN = 2048
EXPERTS = 64
BLOCK = 1024
STEPS = NUM_TOKENS // BLOCK
OROWS = BLOCK // 128
KHALF = HIDDEN // 2


def _router_block(xl_ref, xr_ref, w_ref, w_out_ref, idx_out_ref):
    w = w_ref[...]
    dims = (((1,), (1,)), ((), ()))
    logits = jax.lax.dot_general(
        xl_ref[...], w[:, :KHALF], dims, preferred_element_type=jnp.float32)
    logits += jax.lax.dot_general(
        xr_ref[...], w[:, KHALF:], dims, preferred_element_type=jnp.float32)
    m = jnp.max(logits, axis=1, keepdims=True)
    s = jnp.sum(jnp.exp(logits - m), axis=1, keepdims=True)
    lane = jax.lax.broadcasted_iota(jnp.int32, logits.shape, 1)
    idx = jnp.min(jnp.where(logits == m, lane, EXPERTS), axis=1, keepdims=True)
    w_out_ref[...] = jnp.reshape(1.0 / s, (OROWS, 128))
    idx_out_ref[...] = jnp.reshape(idx, (OROWS, 128))


@functools.partial(jax.jit, static_argnames=())
def kernel(hidden_states, W_gate):
    weights, indices = pl.pallas_call(
        _router_block,
        grid=(STEPS,),
        in_specs=[
            pl.BlockSpec((BLOCK, KHALF), lambda i: (i, 0)),
            pl.BlockSpec((BLOCK, KHALF), lambda i: (i, 1)),
            pl.BlockSpec((EXPERTS, HIDDEN), lambda i: (0, 0)),
        ],
        out_specs=[
            pl.BlockSpec((OROWS, 128), lambda i: (i, 0)),
            pl.BlockSpec((OROWS, 128), lambda i: (i, 0)),
        ],
        out_shape=[
            jax.ShapeDtypeStruct((NUM_TOKENS // 128, 128), jnp.float32),
            jax.ShapeDtypeStruct((NUM_TOKENS // 128, 128), jnp.int32),
        ],
    )(hidden_states, hidden_states, W_gate)
    return (weights.reshape(NUM_TOKENS, 1),
            indices.reshape(NUM_TOKENS, 1).astype(jnp.int64))


# speedup vs baseline: 1.0142x; 1.0142x over previous
import functools

import jax
import jax.numpy as jnp
from jax.experimental import pallas as pl

NUM_TOKENS = 16384
HIDDEN = 2048
EXPERTS = 64
BLOCK = 2048
STEPS = NUM_TOKENS // BLOCK
OROWS = BLOCK // 128
KHALF = HIDDEN // 2


def _router_block(xl_ref, xr_ref, w_ref, w_out_ref, idx_out_ref):
    w = w_ref[...]
    dims = (((1,), (1,)), ((), ()))
    logits = jax.lax.dot_general(
        xl_ref[...], w[:, :KHALF], dims, preferred_element_type=jnp.float32)
    logits += jax.lax.dot_general(
        xr_ref[...], w[:, KHALF:], dims, preferred_element_type=jnp.float32)
    m = jnp.max(logits, axis=1, keepdims=True)
    s = jnp.sum(jnp.exp(logits - m), axis=1, keepdims=True)
    lane = jax.lax.broadcasted_iota(jnp.int32, logits.shape, 1)
    idx = jnp.min(jnp.where(logits == m, lane, EXPERTS), axis=1, keepdims=True)
    w_out_ref[...] = jnp.reshape(1.0 / s, (OROWS, 128))
    idx_out_ref[...] = jnp.reshape(idx, (OROWS, 128))


@functools.partial(jax.jit, static_argnames=())
def kernel(hidden_states, W_gate):
    weights, indices = pl.pallas_call(
        _router_block,
        grid=(STEPS,),
        in_specs=[
            pl.BlockSpec((BLOCK, KHALF), lambda i: (i, 0)),
            pl.BlockSpec((BLOCK, KHALF), lambda i: (i, 1)),
            pl.BlockSpec((EXPERTS, HIDDEN), lambda i: (0, 0)),
        ],
        out_specs=[
            pl.BlockSpec((OROWS, 128), lambda i: (i, 0)),
            pl.BlockSpec((OROWS, 128), lambda i: (i, 0)),
        ],
        out_shape=[
            jax.ShapeDtypeStruct((NUM_TOKENS // 128, 128), jnp.float32),
            jax.ShapeDtypeStruct((NUM_TOKENS // 128, 128), jnp.int32),
        ],
    )(hidden_states, hidden_states, W_gate)
    return (weights.reshape(NUM_TOKENS, 1),
            indices.reshape(NUM_TOKENS, 1).astype(jnp.int64))
